# baseline (speedup 1.0000x reference)
import functools

import jax
import jax.numpy as jnp
from jax.experimental import pallas as pl
from jax.experimental.pallas import tpu as pltpu
from jax.experimental.pallas import tpu_sc as plsc

_D = 128
_B = 16
_K = 8
_HIGH = jax.lax.Precision.HIGHEST


def _row_tile(n):
    for t in (400, 200, 100, 50, 8):
        if n % t == 0:
            return t
    return n


def _prep_body(x_ref, wa_ref, wb_ref, b1_ref, xn_ref, z_ref, u_ref, sq_ref):
    x = x_ref[...]
    nrm = jnp.sqrt(jnp.sum(x * x, axis=1, keepdims=True))
    xn = x / jnp.clip(nrm, 1e-12, None)
    xn_ref[...] = xn
    sq_ref[...] = jnp.sum(xn * xn, axis=1, keepdims=True)
    z_ref[...] = jnp.dot(xn, wb_ref[...],
                         preferred_element_type=jnp.float32)
    u_ref[...] = jnp.dot(xn, wa_ref[...] - wb_ref[...],
                         preferred_element_type=jnp.float32) + b1_ref[...]


def _prep(x, w1, b1, t):
    n, d = x.shape
    grid = n // t
    return pl.pallas_call(
        _prep_body,
        grid=(grid,),
        in_specs=[
            pl.BlockSpec((t, d), lambda i: (i, 0)),
            pl.BlockSpec((d, d), lambda i: (0, 0)),
            pl.BlockSpec((d, d), lambda i: (0, 0)),
            pl.BlockSpec((1, d), lambda i: (0, 0)),
        ],
        out_specs=[
            pl.BlockSpec((t, d), lambda i: (i, 0)),
            pl.BlockSpec((t, d), lambda i: (i, 0)),
            pl.BlockSpec((t, d), lambda i: (i, 0)),
            pl.BlockSpec((t, 1), lambda i: (i, 0)),
        ],
        out_shape=[jax.ShapeDtypeStruct((n, d), jnp.float32)] * 3
        + [jax.ShapeDtypeStruct((n, 1), jnp.float32)],
    )(x, w1[:d], w1[d:], b1.reshape(1, d))


_BIGID = 16777216.0


def _knn_body(lohi_ref, xt_ref, rb_ref, xc_ref, bc_ref, sqc_ref, idx_ref,
              *, t, c):
    i = pl.program_id(0)
    xi = xt_ref[...]
    rb = rb_ref[...]
    sqi = jnp.sum(xi * xi, axis=1, keepdims=True)
    c0 = lohi_ref[0, i]
    c1 = lohi_ref[1, i]

    def chunk(ci, carry):
        vals, ids = carry
        xc = xc_ref[ci]
        bc = bc_ref[ci]
        dp = jax.lax.dot_general(xi, xc, (((1,), (1,)), ((), ())),
                                 preferred_element_type=jnp.float32)
        d2 = sqi + sqc_ref[ci] - 2.0 * dp
        d2 = d2 + jnp.where(rb != bc, jnp.float32(1e10), jnp.float32(0.0))
        cols = ((ci * c).astype(jnp.float32)
                + jax.lax.broadcasted_iota(jnp.int32, d2.shape, 1)
                .astype(jnp.float32))
        av = jnp.concatenate([vals, d2], axis=1)
        ai = jnp.concatenate([ids, cols], axis=1)
        vs, js = [], []
        for _ in range(_K):
            m = jnp.min(av, axis=1, keepdims=True)
            sel = jnp.min(jnp.where(av == m, ai, jnp.float32(_BIGID)),
                          axis=1, keepdims=True)
            vs.append(m)
            js.append(sel)
            av = jnp.where(ai == sel, jnp.float32(jnp.inf), av)
        return jnp.concatenate(vs, axis=1), jnp.concatenate(js, axis=1)

    init = (jnp.full((t, _K), jnp.inf, jnp.float32),
            jnp.zeros((t, _K), jnp.float32))
    _, ids = jax.lax.fori_loop(c0, c1, chunk, init)
    idx_ref[...] = ids.astype(jnp.int32)


def _knn(lohi, xn, rb2, xc3, bc3, sqc3, t, c):
    n, d = xn.shape
    nc = xc3.shape[0]
    grid_spec = pltpu.PrefetchScalarGridSpec(
        num_scalar_prefetch=1,
        grid=(n // t,),
        in_specs=[
            pl.BlockSpec((t, d), lambda i, s: (i, 0)),
            pl.BlockSpec((t, 1), lambda i, s: (i, 0)),
            pl.BlockSpec((nc, c, d), lambda i, s: (0, 0, 0)),
            pl.BlockSpec((nc, 1, c), lambda i, s: (0, 0, 0)),
            pl.BlockSpec((nc, 1, c), lambda i, s: (0, 0, 0)),
        ],
        out_specs=pl.BlockSpec((t, _K), lambda i, s: (i, 0)),
    )
    return pl.pallas_call(
        functools.partial(_knn_body, t=t, c=c),
        grid_spec=grid_spec,
        out_shape=jax.ShapeDtypeStruct((n, _K), jnp.int32),
    )(lohi, xn, rb2, xc3, bc3, sqc3)


def _sc_gather(z, idx_flat, gw):
    edge = idx_flat.shape[1]
    d = z.shape[1]
    mesh = plsc.VectorSubcoreMesh(core_axis_name="core",
                                  subcore_axis_name="subcore")

    @pl.kernel(out_type=jax.ShapeDtypeStruct((edge, d), jnp.float32),
               mesh=mesh)
    def gather_kernel(z_hbm, i_hbm, o_hbm):
        def body(i_vmem, o_vmem):
            pltpu.sync_copy(z_hbm.at[i_vmem.at[0]], o_vmem)

        pltpu.emit_pipeline(
            body,
            grid=(edge // gw,),
            in_specs=[pl.BlockSpec((1, gw), index_map=lambda i: (0, i))],
            out_specs=[pl.BlockSpec((gw, d), index_map=lambda i: (i, 0))],
            core_axis_name=("core", "subcore"),
            dimension_semantics=(pltpu.PARALLEL,),
        )(i_hbm, o_hbm)

    return gather_kernel(z, idx_flat)


def _stats_body(u_ref, g_ref, s1_ref, s2_ref):
    i = pl.program_id(0)

    @pl.when(i == 0)
    def _():
        s1_ref[...] = jnp.zeros_like(s1_ref)
        s2_ref[...] = jnp.zeros_like(s2_ref)

    u = u_ref[...]
    a1 = jnp.zeros((1, u.shape[1]), jnp.float32)
    a2 = jnp.zeros((1, u.shape[1]), jnp.float32)
    for k in range(_K):
        hk = u + g_ref[k]
        a1 = a1 + jnp.sum(hk, axis=0, keepdims=True)
        a2 = a2 + jnp.sum(hk * hk, axis=0, keepdims=True)
    s1_ref[0:1, :] += a1
    s2_ref[0:1, :] += a2


def _stats(u, g3, t):
    n, d = u.shape
    return pl.pallas_call(
        _stats_body,
        grid=(n // t,),
        in_specs=[
            pl.BlockSpec((t, d), lambda i: (i, 0)),
            pl.BlockSpec((_K, t, d), lambda i: (0, i, 0)),
        ],
        out_specs=[
            pl.BlockSpec((8, d), lambda i: (0, 0)),
            pl.BlockSpec((8, d), lambda i: (0, 0)),
        ],
        out_shape=[jax.ShapeDtypeStruct((8, d), jnp.float32)] * 2,
    )(u, g3)


def _edge_body(u_ref, g_ref, rb_ref, s1_ref, s2_ref, g1_ref, be1_ref,
               w2_ref, b2_ref, pooled_ref, *, t, nk):
    i = pl.program_id(0)

    @pl.when(i == 0)
    def _():
        pooled_ref[...] = jnp.full_like(pooled_ref, -jnp.inf)

    d = u_ref.shape[1]
    mu = jnp.sum(s1_ref[...], axis=0, keepdims=True) / nk
    msq = jnp.sum(s2_ref[...], axis=0, keepdims=True) / nk
    var = msq - mu * mu
    scale = g1_ref[...] / jnp.sqrt(var + 1e-5)
    shift = be1_ref[...] - mu * scale

    u = u_ref[...]
    w2 = w2_ref[...]
    node = None
    for k in range(_K):
        rk = jnp.maximum((u + g_ref[k]) * scale + shift, 0.0)
        ek = jnp.dot(rk, w2, preferred_element_type=jnp.float32)
        node = ek if node is None else jnp.maximum(node, ek)
    node = node + b2_ref[...]

    rb = rb_ref[...]
    conts = [jnp.max(jnp.where(rb == b, node, -jnp.inf), axis=0,
                     keepdims=True) for b in range(_B)]
    pooled_ref[...] = jnp.maximum(pooled_ref[...],
                                  jnp.concatenate(conts, axis=0))


def _edge(u, g, rb2, s1, s2, g1, be1, w2, b2, t):
    n, d = u.shape
    return pl.pallas_call(
        functools.partial(_edge_body, t=t, nk=float(n * _K)),
        grid=(n // t,),
        in_specs=[
            pl.BlockSpec((t, d), lambda i: (i, 0)),
            pl.BlockSpec((_K, t, d), lambda i: (0, i, 0)),
            pl.BlockSpec((t, 1), lambda i: (i, 0)),
            pl.BlockSpec((8, d), lambda i: (0, 0)),
            pl.BlockSpec((8, d), lambda i: (0, 0)),
            pl.BlockSpec((1, d), lambda i: (0, 0)),
            pl.BlockSpec((1, d), lambda i: (0, 0)),
            pl.BlockSpec((d, d), lambda i: (0, 0)),
            pl.BlockSpec((1, d), lambda i: (0, 0)),
        ],
        out_specs=pl.BlockSpec((_B, d), lambda i: (0, 0)),
        out_shape=jax.ShapeDtypeStruct((_B, d), jnp.float32),
    )(u, g, rb2, s1, s2, g1.reshape(1, d), be1.reshape(1, d), w2,
      b2.reshape(1, d))


def _final_body(p_ref, wl1_ref, bl1_ref, wl2_ref, bl2_ref, o_ref):
    p = jnp.dot(p_ref[...], wl1_ref[...], precision=_HIGH,
                preferred_element_type=jnp.float32) + bl1_ref[...]
    p = jnp.maximum(p, 0.0)
    o = jnp.dot(p, wl2_ref[...], precision=_HIGH,
                preferred_element_type=jnp.float32) + bl2_ref[...]
    nrm = jnp.sqrt(jnp.sum(o * o, axis=1, keepdims=True))
    o_ref[...] = o / jnp.clip(nrm, 1e-12, None)


def _final(pooled, wl1, bl1, wl2, bl2):
    d = pooled.shape[1]
    return pl.pallas_call(
        _final_body,
        out_shape=jax.ShapeDtypeStruct((_B, d), jnp.float32),
    )(pooled, wl1, bl1.reshape(1, d), wl2, bl2.reshape(1, d))


def _run(embeddings, batch, W1, b1, g1, be1, W2, b2, Wl1, bl1, Wl2, bl2,
         gather_fn):
    n, d = embeddings.shape
    t = _row_tile(n)
    c = 512
    npad = ((n + c - 1) // c) * c
    nc = npad // c

    tk = t // 2 if n % (t // 2) == 0 else t
    b32 = batch.astype(jnp.int32)
    starts = jnp.searchsorted(b32, jnp.arange(_B + 1, dtype=jnp.int32),
                              side="left").astype(jnp.int32)
    tix = jnp.arange(n // tk, dtype=jnp.int32)
    lo = starts[b32[tix * tk]]
    hi = starts[b32[tix * tk + (tk - 1)] + 1]
    lohi = jnp.stack([lo // c, (hi + c - 1) // c]).astype(jnp.int32)

    xn, z, u, sqn = _prep(embeddings, W1, b1, t)

    xc3 = jnp.pad(xn, ((0, npad - n), (0, 0))).reshape(nc, c, d)
    bc3 = jnp.pad(b32, (0, npad - n),
                  constant_values=_B).reshape(nc, 1, c)
    sqc3 = jnp.pad(sqn.reshape(n), (0, npad - n)).reshape(nc, 1, c)
    rb2 = b32.reshape(n, 1)

    idx = _knn(lohi, xn, rb2, xc3, bc3, sqc3, tk, c)
    g3 = gather_fn(z, idx.T.reshape(1, n * _K)).reshape(_K, n, d)

    s1, s2 = _stats(u, g3, t)
    pooled = _edge(u, g3, rb2, s1, s2, g1, be1, W2, b2, t)
    return _final(pooled, Wl1, bl1, Wl2, bl2)


def kernel(embeddings, batch, W1, b1, g1, be1, W2, b2, Wl1, bl1, Wl2, bl2):
    return _run(embeddings, batch, W1, b1, g1, be1, W2, b2,
                Wl1, bl1, Wl2, bl2,
                functools.partial(_sc_gather, gw=128))


# speedup vs baseline: 1.1938x; 1.1938x over previous
import functools

import jax
import jax.numpy as jnp
from jax.experimental import pallas as pl
from jax.experimental.pallas import tpu as pltpu
from jax.experimental.pallas import tpu_sc as plsc

_D = 128
_B = 16
_K = 8
_HIGH = jax.lax.Precision.HIGHEST


def _row_tile(n):
    for t in (400, 200, 100, 50, 8):
        if n % t == 0:
            return t
    return n


def _prep_body(x_ref, wa_ref, wb_ref, b1_ref, xn_ref, z_ref, u_ref, sq_ref):
    x = x_ref[...]
    nrm = jnp.sqrt(jnp.sum(x * x, axis=1, keepdims=True))
    xn = x / jnp.clip(nrm, 1e-12, None)
    xn_ref[...] = xn
    sq_ref[...] = jnp.sum(xn * xn, axis=1, keepdims=True)
    z_ref[...] = jnp.dot(xn, wb_ref[...],
                         preferred_element_type=jnp.float32)
    u_ref[...] = jnp.dot(xn, wa_ref[...] - wb_ref[...],
                         preferred_element_type=jnp.float32) + b1_ref[...]


def _prep(x, w1, b1, t):
    n, d = x.shape
    grid = n // t
    return pl.pallas_call(
        _prep_body,
        grid=(grid,),
        in_specs=[
            pl.BlockSpec((t, d), lambda i: (i, 0)),
            pl.BlockSpec((d, d), lambda i: (0, 0)),
            pl.BlockSpec((d, d), lambda i: (0, 0)),
            pl.BlockSpec((1, d), lambda i: (0, 0)),
        ],
        out_specs=[
            pl.BlockSpec((t, d), lambda i: (i, 0)),
            pl.BlockSpec((t, d), lambda i: (i, 0)),
            pl.BlockSpec((t, d), lambda i: (i, 0)),
            pl.BlockSpec((t, 1), lambda i: (i, 0)),
        ],
        out_shape=[jax.ShapeDtypeStruct((n, d), jnp.float32)] * 3
        + [jax.ShapeDtypeStruct((n, 1), jnp.float32)],
    )(x, w1[:d], w1[d:], b1.reshape(1, d))


_BIGID = 16777216.0


def _knn_body(lohi_ref, xt_ref, rb_ref, xc_ref, bc_ref, sqc_ref, idx_ref,
              *, t, c):
    i = pl.program_id(0)
    xi = xt_ref[...]
    rb = rb_ref[...]
    sqi = jnp.sum(xi * xi, axis=1, keepdims=True)
    c0 = lohi_ref[0, i]
    c1 = lohi_ref[1, i]

    def chunk(ci, carry):
        vals, ids = carry
        xc = xc_ref[ci]
        bc = bc_ref[ci]
        dp = jax.lax.dot_general(xi, xc, (((1,), (1,)), ((), ())),
                                 preferred_element_type=jnp.float32)
        d2 = sqi + sqc_ref[ci] - 2.0 * dp
        d2 = d2 + jnp.where(rb != bc, jnp.float32(1e10), jnp.float32(0.0))
        cols = ((ci * c).astype(jnp.float32)
                + jax.lax.broadcasted_iota(jnp.int32, d2.shape, 1)
                .astype(jnp.float32))
        av = jnp.concatenate([vals, d2], axis=1)
        ai = jnp.concatenate([ids, cols], axis=1)
        vs, js = [], []
        for _ in range(_K):
            m = jnp.min(av, axis=1, keepdims=True)
            sel = jnp.min(jnp.where(av == m, ai, jnp.float32(_BIGID)),
                          axis=1, keepdims=True)
            vs.append(m)
            js.append(sel)
            av = jnp.where(ai == sel, jnp.float32(jnp.inf), av)
        return jnp.concatenate(vs, axis=1), jnp.concatenate(js, axis=1)

    init = (jnp.full((t, _K), jnp.inf, jnp.float32),
            jnp.zeros((t, _K), jnp.float32))
    _, ids = jax.lax.fori_loop(c0, c1, chunk, init)
    idx_ref[...] = ids.astype(jnp.int32)


def _knn(lohi, xn, rb2, xc3, bc3, sqc3, t, c):
    n, d = xn.shape
    nc = xc3.shape[0]
    grid_spec = pltpu.PrefetchScalarGridSpec(
        num_scalar_prefetch=1,
        grid=(n // t,),
        in_specs=[
            pl.BlockSpec((t, d), lambda i, s: (i, 0)),
            pl.BlockSpec((t, 1), lambda i, s: (i, 0)),
            pl.BlockSpec((nc, c, d), lambda i, s: (0, 0, 0)),
            pl.BlockSpec((nc, 1, c), lambda i, s: (0, 0, 0)),
            pl.BlockSpec((nc, 1, c), lambda i, s: (0, 0, 0)),
        ],
        out_specs=pl.BlockSpec((t, _K), lambda i, s: (i, 0)),
    )
    return pl.pallas_call(
        functools.partial(_knn_body, t=t, c=c),
        grid_spec=grid_spec,
        out_shape=jax.ShapeDtypeStruct((n, _K), jnp.int32),
    )(lohi, xn, rb2, xc3, bc3, sqc3)


def _sc_gather(z, idx_flat, gw):
    edge = idx_flat.shape[1]
    d = z.shape[1]
    mesh = plsc.VectorSubcoreMesh(core_axis_name="core",
                                  subcore_axis_name="subcore")

    @pl.kernel(out_type=jax.ShapeDtypeStruct((edge, d), jnp.float32),
               mesh=mesh)
    def gather_kernel(z_hbm, i_hbm, o_hbm):
        def body(i_vmem, o_vmem):
            pltpu.sync_copy(z_hbm.at[i_vmem.at[0]], o_vmem)

        pltpu.emit_pipeline(
            body,
            grid=(edge // gw,),
            in_specs=[pl.BlockSpec((1, gw), index_map=lambda i: (0, i))],
            out_specs=[pl.BlockSpec((gw, d), index_map=lambda i: (i, 0))],
            core_axis_name=("core", "subcore"),
            dimension_semantics=(pltpu.PARALLEL,),
        )(i_hbm, o_hbm)

    return gather_kernel(z, idx_flat)


def _stats_body(u_ref, g_ref, s1_ref, s2_ref):
    i = pl.program_id(0)

    @pl.when(i == 0)
    def _():
        s1_ref[...] = jnp.zeros_like(s1_ref)
        s2_ref[...] = jnp.zeros_like(s2_ref)

    u = u_ref[...]
    a1 = jnp.zeros((1, u.shape[1]), jnp.float32)
    a2 = jnp.zeros((1, u.shape[1]), jnp.float32)
    for k in range(_K):
        hk = u + g_ref[k]
        a1 = a1 + jnp.sum(hk, axis=0, keepdims=True)
        a2 = a2 + jnp.sum(hk * hk, axis=0, keepdims=True)
    s1_ref[0:1, :] += a1
    s2_ref[0:1, :] += a2


def _stats(u, g3, t):
    n, d = u.shape
    return pl.pallas_call(
        _stats_body,
        grid=(n // t,),
        in_specs=[
            pl.BlockSpec((t, d), lambda i: (i, 0)),
            pl.BlockSpec((_K, t, d), lambda i: (0, i, 0)),
        ],
        out_specs=[
            pl.BlockSpec((8, d), lambda i: (0, 0)),
            pl.BlockSpec((8, d), lambda i: (0, 0)),
        ],
        out_shape=[jax.ShapeDtypeStruct((8, d), jnp.float32)] * 2,
    )(u, g3)


def _edge_body(u_ref, g_ref, rb_ref, s1_ref, s2_ref, g1_ref, be1_ref,
               w2_ref, b2_ref, wl1_ref, bl1_ref, wl2_ref, bl2_ref,
               out_ref, pooled_ref, *, t, nk):
    i = pl.program_id(0)

    @pl.when(i == 0)
    def _():
        pooled_ref[...] = jnp.full_like(pooled_ref, -jnp.inf)

    d = u_ref.shape[1]
    mu = jnp.sum(s1_ref[...], axis=0, keepdims=True) / nk
    msq = jnp.sum(s2_ref[...], axis=0, keepdims=True) / nk
    var = msq - mu * mu
    scale = g1_ref[...] / jnp.sqrt(var + 1e-5)
    shift = be1_ref[...] - mu * scale

    u = u_ref[...]
    w2 = w2_ref[...]
    node = None
    for k in range(_K):
        rk = jnp.maximum((u + g_ref[k]) * scale + shift, 0.0)
        ek = jnp.dot(rk, w2, preferred_element_type=jnp.float32)
        node = ek if node is None else jnp.maximum(node, ek)
    node = node + b2_ref[...]

    rb = rb_ref[...]
    conts = [jnp.max(jnp.where(rb == b, node, -jnp.inf), axis=0,
                     keepdims=True) for b in range(_B)]
    pooled_ref[...] = jnp.maximum(pooled_ref[...],
                                  jnp.concatenate(conts, axis=0))

    @pl.when(i == pl.num_programs(0) - 1)
    def _():
        p = jnp.dot(pooled_ref[...], wl1_ref[...],
                    preferred_element_type=jnp.float32) + bl1_ref[...]
        p = jnp.maximum(p, 0.0)
        o = jnp.dot(p, wl2_ref[...],
                    preferred_element_type=jnp.float32) + bl2_ref[...]
        nrm = jnp.sqrt(jnp.sum(o * o, axis=1, keepdims=True))
        out_ref[...] = o / jnp.clip(nrm, 1e-12, None)


def _edge(u, g, rb2, s1, s2, g1, be1, w2, b2, wl1, bl1, wl2, bl2, t):
    n, d = u.shape
    return pl.pallas_call(
        functools.partial(_edge_body, t=t, nk=float(n * _K)),
        grid=(n // t,),
        in_specs=[
            pl.BlockSpec((t, d), lambda i: (i, 0)),
            pl.BlockSpec((_K, t, d), lambda i: (0, i, 0)),
            pl.BlockSpec((t, 1), lambda i: (i, 0)),
            pl.BlockSpec((8, d), lambda i: (0, 0)),
            pl.BlockSpec((8, d), lambda i: (0, 0)),
            pl.BlockSpec((1, d), lambda i: (0, 0)),
            pl.BlockSpec((1, d), lambda i: (0, 0)),
            pl.BlockSpec((d, d), lambda i: (0, 0)),
            pl.BlockSpec((1, d), lambda i: (0, 0)),
            pl.BlockSpec((d, d), lambda i: (0, 0)),
            pl.BlockSpec((1, d), lambda i: (0, 0)),
            pl.BlockSpec((d, d), lambda i: (0, 0)),
            pl.BlockSpec((1, d), lambda i: (0, 0)),
        ],
        out_specs=pl.BlockSpec((_B, d), lambda i: (0, 0)),
        out_shape=jax.ShapeDtypeStruct((_B, d), jnp.float32),
        scratch_shapes=[pltpu.VMEM((_B, d), jnp.float32)],
    )(u, g, rb2, s1, s2, g1.reshape(1, d), be1.reshape(1, d), w2,
      b2.reshape(1, d), wl1, bl1.reshape(1, d), wl2, bl2.reshape(1, d))


def _run(embeddings, batch, W1, b1, g1, be1, W2, b2, Wl1, bl1, Wl2, bl2,
         gather_fn):
    n, d = embeddings.shape
    t = _row_tile(n)
    c = 512
    npad = ((n + c - 1) // c) * c
    nc = npad // c

    tk = t
    b32 = batch.astype(jnp.int32)
    starts = jnp.searchsorted(b32, jnp.arange(_B + 1, dtype=jnp.int32),
                              side="left").astype(jnp.int32)
    tix = jnp.arange(n // tk, dtype=jnp.int32)
    lo = starts[b32[tix * tk]]
    hi = starts[b32[tix * tk + (tk - 1)] + 1]
    lohi = jnp.stack([lo // c, (hi + c - 1) // c]).astype(jnp.int32)

    xn, z, u, sqn = _prep(embeddings, W1, b1, t)

    xc3 = jnp.pad(xn, ((0, npad - n), (0, 0))).reshape(nc, c, d)
    bc3 = jnp.pad(b32, (0, npad - n),
                  constant_values=_B).reshape(nc, 1, c)
    sqc3 = jnp.pad(sqn.reshape(n), (0, npad - n)).reshape(nc, 1, c)
    rb2 = b32.reshape(n, 1)

    idx = _knn(lohi, xn, rb2, xc3, bc3, sqc3, tk, c)
    g3 = gather_fn(z, idx.T.reshape(1, n * _K)).reshape(_K, n, d)

    s1, s2 = _stats(u, g3, t)
    return _edge(u, g3, rb2, s1, s2, g1, be1, W2, b2,
                 Wl1, bl1, Wl2, bl2, t)


def kernel(embeddings, batch, W1, b1, g1, be1, W2, b2, Wl1, bl1, Wl2, bl2):
    return _run(embeddings, batch, W1, b1, g1, be1, W2, b2,
                Wl1, bl1, Wl2, bl2,
                functools.partial(_sc_gather, gw=128))
